# Initial kernel scaffold; baseline (speedup 1.0000x reference)
import jax
import jax.numpy as jnp
from jax.experimental import pallas as pl


def kernel(features, W0, W1):
    raise NotImplementedError("write your pallas kernel here")


# speedup vs baseline: 28.1454x; 28.1454x over previous
import jax
import jax.numpy as jnp
from jax.experimental import pallas as pl

_N = 8192
_D = 256
_BLK = 4096
_K = 21
_RT = 256
_ET = 1024


def _emb_kernel(f_ref, w0_ref, w1_ref, emb_ref):
    h = jnp.maximum(f_ref[...] * w0_ref[...], 0.0) * w1_ref[...]
    n = jnp.sqrt(jnp.sum(h * h, axis=1, keepdims=True))
    emb_ref[...] = h / jnp.maximum(n, 1e-12)


def _sim_kernel(rows_ref, cols_ref, out_ref):
    a = pl.program_id(0)
    sim = jax.lax.dot_general(
        rows_ref[...], cols_ref[...],
        (((1,), (1,)), ((), ())),
        preferred_element_type=jnp.float32,
    )
    x = sim
    thr = None
    for _ in range(_K):
        thr = jnp.max(x, axis=1, keepdims=True)
        x = jnp.where(x >= thr, -jnp.inf, x)
    masked = jnp.where((sim >= thr) & (sim > 0.0), sim, 0.0)
    out_ref[...] = jnp.zeros_like(out_ref)
    out_ref[:, pl.ds(a * _BLK, _BLK)] = masked


def kernel(features, W0, W1):
    w0 = W0.reshape(1, _D)
    w1 = W1.reshape(1, _D)
    emb = pl.pallas_call(
        _emb_kernel,
        grid=(_N // _ET,),
        in_specs=[
            pl.BlockSpec((_ET, _D), lambda i: (i, 0)),
            pl.BlockSpec((1, _D), lambda i: (0, 0)),
            pl.BlockSpec((1, _D), lambda i: (0, 0)),
        ],
        out_specs=pl.BlockSpec((_ET, _D), lambda i: (i, 0)),
        out_shape=jax.ShapeDtypeStruct((_N, _D), jnp.float32),
    )(features, w0, w1)

    nt = _BLK // _RT
    out = pl.pallas_call(
        _sim_kernel,
        grid=(2, nt),
        in_specs=[
            pl.BlockSpec((_RT, _D), lambda a, i: (a * nt + i, 0)),
            pl.BlockSpec((_BLK, _D), lambda a, i: (a, 0)),
        ],
        out_specs=pl.BlockSpec((_RT, _N), lambda a, i: (a * nt + i, 0)),
        out_shape=jax.ShapeDtypeStruct((_N, _N), jnp.float32),
    )(emb, emb)
    return out
